# baseline (speedup 1.0000x reference)
import functools

import jax
import jax.numpy as jnp
from jax import lax
from jax.experimental import pallas as pl
from jax.experimental.pallas import tpu as pltpu
from jax.experimental.pallas import tpu_sc as plsc

N = 10000
E = 320000
D = 128
L = 4

NC = 2
NS = 16
NW = NC * NS
CHUNK = 128
NCHUNK = E // CHUNK
CPW = 80
NCHUNK_PAD = CPW * NW
HCH = NCHUNK_PAD // 2
EH = HCH * CHUNK
CPWH = HCH // NW
CPTH = HCH // NS
RPT = 624
TAIL = N - RPT * NS
ZR = 208

def _sc_mesh():
    return plsc.VectorSubcoreMesh(
        core_axis_name="c", subcore_axis_name="s", num_cores=NC, num_subcores=NS)

BN = 2000
BE = 2560


def _silu(v):
    return v * jax.nn.sigmoid(v)


def _full_spec(shape):
    return pl.BlockSpec(shape, lambda i: tuple(0 for _ in shape))


def _tc_prologue(h, W_in, b_in, A0, be10, B0):
    def body(h_ref, win_ref, bin_ref, a_ref, be_ref, b_ref, x_ref, xa_ref, xb_ref):
        x = jnp.dot(h_ref[...], win_ref[...], preferred_element_type=jnp.float32)
        x = x + bin_ref[...]
        x_ref[...] = x
        xa_ref[...] = jnp.dot(x, a_ref[...], preferred_element_type=jnp.float32) + be_ref[...]
        xb_ref[...] = jnp.dot(x, b_ref[...], preferred_element_type=jnp.float32)

    grid = (N // BN,)
    blk = pl.BlockSpec((BN, D), lambda i: (i, 0))
    return pl.pallas_call(
        body,
        grid=grid,
        in_specs=[blk, _full_spec((D, D)), _full_spec((1, D)),
                  _full_spec((D, D)), _full_spec((1, D)), _full_spec((D, D))],
        out_specs=[blk, blk, blk],
        out_shape=[jax.ShapeDtypeStruct((N, D), jnp.float32)] * 3,
    )(h, W_in, b_in, A0, be10, B0)


def _tc_edge(srcA, dstB, cd, w_r, We2, be2):
    def body(sa_ref, db_ref, cd_ref, wr_ref, w2_ref, b2_ref, m_ref):
        c = cd_ref[...]
        radial = jnp.sum(c * c, axis=1, keepdims=True)
        pre = sa_ref[...] + db_ref[...] + radial * wr_ref[...]
        m1 = _silu(pre)
        m_ref[...] = _silu(
            jnp.dot(m1, w2_ref[...], preferred_element_type=jnp.float32) + b2_ref[...])

    grid = (EH // BE,)
    blk = pl.BlockSpec((BE, D), lambda i: (i, 0))
    return pl.pallas_call(
        body,
        grid=grid,
        in_specs=[blk, blk, pl.BlockSpec((BE, 3), lambda i: (i, 0)),
                  _full_spec((1, D)), _full_spec((D, D)), _full_spec((1, D))],
        out_specs=blk,
        out_shape=jax.ShapeDtypeStruct((EH, D), jnp.float32),
    )(srcA, dstB, cd, w_r, We2, be2)


def _tc_node(x, a0, a1, Wn1x, Wn1a, bn1, Wn2, bn2, An, ben, Bn):
    def body(x_ref, a0_ref, a1_ref, w1x_ref, w1a_ref, b1_ref, w2_ref, b2_ref,
             an_ref, ben_ref, bn_ref, xn_ref, xa_ref, xb_ref):
        x = x_ref[...]
        agg = a0_ref[...] + a1_ref[...]
        hmid = _silu(
            jnp.dot(x, w1x_ref[...], preferred_element_type=jnp.float32)
            + jnp.dot(agg, w1a_ref[...], preferred_element_type=jnp.float32)
            + b1_ref[...])
        xn = jnp.dot(hmid, w2_ref[...], preferred_element_type=jnp.float32) + b2_ref[...]
        xn_ref[...] = xn
        xa_ref[...] = jnp.dot(xn, an_ref[...], preferred_element_type=jnp.float32) + ben_ref[...]
        xb_ref[...] = jnp.dot(xn, bn_ref[...], preferred_element_type=jnp.float32)

    grid = (N // BN,)
    blk = pl.BlockSpec((BN, D), lambda i: (i, 0))
    return pl.pallas_call(
        body,
        grid=grid,
        in_specs=[blk, blk, blk,
                  _full_spec((D, D)), _full_spec((D, D)), _full_spec((1, D)),
                  _full_spec((D, D)), _full_spec((1, D)),
                  _full_spec((D, D)), _full_spec((1, D)), _full_spec((D, D))],
        out_specs=[blk, blk, blk],
        out_shape=[jax.ShapeDtypeStruct((N, D), jnp.float32)] * 3,
    )(x, a0, a1, Wn1x, Wn1a, bn1, Wn2, bn2, An, ben, Bn)


def _tc_node_last(x, a0, a1, Wn1x, Wn1a, bn1, Wn2, bn2, W_out, b_out):
    def body(x_ref, a0_ref, a1_ref, w1x_ref, w1a_ref, b1_ref, w2_ref, b2_ref,
             wo_ref, bo_ref, o_ref):
        x = x_ref[...]
        agg = a0_ref[...] + a1_ref[...]
        hmid = _silu(
            jnp.dot(x, w1x_ref[...], preferred_element_type=jnp.float32)
            + jnp.dot(agg, w1a_ref[...], preferred_element_type=jnp.float32)
            + b1_ref[...])
        xn = jnp.dot(hmid, w2_ref[...], preferred_element_type=jnp.float32) + b2_ref[...]
        o_ref[...] = jnp.dot(xn, wo_ref[...], preferred_element_type=jnp.float32) + bo_ref[...]

    grid = (N // BN,)
    blk = pl.BlockSpec((BN, D), lambda i: (i, 0))
    return pl.pallas_call(
        body,
        grid=grid,
        in_specs=[blk, blk, blk,
                  _full_spec((D, D)), _full_spec((D, D)), _full_spec((1, D)),
                  _full_spec((D, D)), _full_spec((1, D)),
                  _full_spec((D, D)), _full_spec((1, D))],
        out_specs=blk,
        out_shape=jax.ShapeDtypeStruct((N, D), jnp.float32),
    )(x, a0, a1, Wn1x, Wn1a, bn1, Wn2, bn2, W_out, b_out)


def _sc_gather(xa, xb, row2, col2):
    @functools.partial(
        pl.kernel,
        out_type=(jax.ShapeDtypeStruct((HCH, CHUNK, D), jnp.float32),
                  jax.ShapeDtypeStruct((HCH, CHUNK, D), jnp.float32)),
        mesh=_sc_mesh(),
        scratch_types=[
            pltpu.VMEM((CPWH, CHUNK), jnp.int32),
            pltpu.VMEM((CPWH, CHUNK), jnp.int32),
            pltpu.VMEM((2, CHUNK, D), jnp.float32),
            pltpu.VMEM((2, CHUNK, D), jnp.float32),
            [pltpu.SemaphoreType.DMA] * 2,
            [pltpu.SemaphoreType.DMA] * 2,
            [pltpu.SemaphoreType.DMA] * 2,
            [pltpu.SemaphoreType.DMA] * 2,
        ],
    )
    def k(xa_h, xb_h, row_h, col_h, srcA_h, dstB_h, idx_r, idx_c,
          bufA, bufB, sgA, sgB, swA, swB):
        w = lax.axis_index("s") * NC + lax.axis_index("c")
        c0 = CPWH * w
        pltpu.sync_copy(row_h.at[pl.ds(c0, CPWH)], idx_r)
        pltpu.sync_copy(col_h.at[pl.ds(c0, CPWH)], idx_c)

        def gath(j, b):
            pltpu.async_copy(xa_h.at[idx_r.at[j]], bufA.at[b], sgA[b])
            pltpu.async_copy(xb_h.at[idx_c.at[j]], bufB.at[b], sgB[b])

        def wait_g(j, b):
            pltpu.make_async_copy(xa_h.at[idx_r.at[j]], bufA.at[b], sgA[b]).wait()
            pltpu.make_async_copy(xb_h.at[idx_c.at[j]], bufB.at[b], sgB[b]).wait()

        def wrb(j, b):
            pltpu.async_copy(bufA.at[b], srcA_h.at[c0 + j], swA[b])
            pltpu.async_copy(bufB.at[b], dstB_h.at[c0 + j], swB[b])

        def wait_w(j, b):
            pltpu.make_async_copy(bufA.at[b], srcA_h.at[c0 + j], swA[b]).wait()
            pltpu.make_async_copy(bufB.at[b], dstB_h.at[c0 + j], swB[b]).wait()

        gath(0, 0)
        gath(1, 1)
        wait_g(0, 0)
        wrb(0, 0)

        def body(p, carry):
            for b2 in range(2):
                kk = 2 * p + b2 + 1
                sb = (b2 + 1) % 2
                so = 1 - sb
                wait_g(kk, sb)
                wrb(kk, sb)
                wait_w(kk - 1, so)

                @pl.when(kk + 1 < CPWH)
                def _():
                    gath(kk + 1, so)
            return carry

        lax.fori_loop(0, (CPWH - 1) // 2, body, 0)
        kk = CPWH - 1
        wait_g(kk, 1)
        wrb(kk, 1)
        wait_w(kk - 1, 0)
        wait_w(kk, 1)

    return k(xa, xb, row2, col2)


HALF0 = 5120
ACC_ROWS = 5136
DUMP = 5128
CPT = 160


def _sc_scatter(m3, row2, nreal):
    @functools.partial(
        pl.kernel,
        out_type=jax.ShapeDtypeStruct((N, D), jnp.float32),
        mesh=_sc_mesh(),
        scratch_types=[
            pltpu.VMEM((CPTH, CHUNK), jnp.int32),
            pltpu.VMEM((CPTH, CHUNK), jnp.int32),
            pltpu.VMEM((2, CHUNK, D), jnp.float32),
            pltpu.VMEM((16, D), jnp.float32),
            pltpu.VMEM_SHARED((ACC_ROWS, D), jnp.float32),
            [pltpu.SemaphoreType.DMA] * 2,
            [pltpu.SemaphoreType.DMA] * 2,
        ],
    )
    def k(m_h, row_h, out_h, idx, idx2, buf, zbuf, acc_sh, sg, sw):
        c = lax.axis_index("c")
        s = lax.axis_index("s")
        base = c * HALF0
        owned = jnp.where(c == 0, HALF0, N - HALF0)
        j0 = CPTH * s
        nj = jnp.clip(nreal - j0, 0, CPTH)
        pltpu.sync_copy(row_h.at[pl.ds(j0, CPTH)], idx)

        zv = jnp.zeros((16,), jnp.float32)
        for r in range(16):
            for g in range(D // 16):
                zbuf[r, g * 16:(g + 1) * 16] = zv

        def remap(j, carry):
            for g in range(D // 16):
                v = idx[j, g * 16:(g + 1) * 16] - base
                ok = (v >= 0) & (v < owned)
                idx2[j, g * 16:(g + 1) * 16] = jnp.where(ok, v, DUMP)
            return carry

        lax.fori_loop(0, nj, remap, 0)

        tb = pl.multiple_of(s * jnp.where(c == 0, HALF0 // NS, 304), 16)
        n16 = jnp.where(c == 0, (HALF0 // NS) // 16, 304 // 16)

        def zero(t, carry):
            off = pl.multiple_of(tb + 16 * t, 16)
            pltpu.sync_copy(zbuf, acc_sh.at[pl.ds(off, 16)])
            return carry

        lax.fori_loop(0, n16, zero, 0)

        @pl.when((c == 1) & (s == NS - 1))
        def _ztail():
            pltpu.sync_copy(zbuf, acc_sh.at[pl.ds(304 * NS, 16)])

        plsc.subcore_barrier()

        def load(j, b):
            pltpu.async_copy(m_h.at[j0 + j], buf.at[b], sg[b])

        def wait_l(j, b):
            pltpu.make_async_copy(m_h.at[j0 + j], buf.at[b], sg[b]).wait()

        def scat(j, b):
            pltpu.async_copy(buf.at[b], acc_sh.at[idx2.at[j]], sw[b], add=True)

        def wait_s(j, b):
            pltpu.make_async_copy(buf.at[b], acc_sh.at[idx2.at[j]], sw[b]).wait()

        load(0, 0)
        load(1, 1)
        wait_l(0, 0)
        scat(0, 0)

        def body(p, carry):
            for b2 in range(2):
                kk = 2 * p + b2 + 1
                sb = (b2 + 1) % 2
                so = 1 - sb
                wait_l(kk, sb)
                scat(kk, sb)
                wait_s(kk - 1, so)

                @pl.when(kk + 1 < nj)
                def _():
                    load(kk + 1, so)
            return carry

        lax.fori_loop(0, (nj - 1) // 2, body, 0)
        kk = nj - 1
        wait_l(kk, 1)
        scat(kk, 1)
        wait_s(kk - 1, 0)
        wait_s(kk, 1)
        plsc.subcore_barrier()

        @pl.when(c == 0)
        def _wb0():
            off = pl.multiple_of((HALF0 // NS) * s, 8)
            pltpu.sync_copy(acc_sh.at[pl.ds(off, HALF0 // NS)],
                            out_h.at[pl.ds(off, HALF0 // NS)])

        @pl.when(c == 1)
        def _wb1():
            off = pl.multiple_of(304 * s, 8)
            pltpu.sync_copy(acc_sh.at[pl.ds(off, 304)],
                            out_h.at[pl.ds(HALF0 + off, 304)])

        @pl.when((c == 1) & (s == NS - 1))
        def _wb1tail():
            pltpu.sync_copy(acc_sh.at[pl.ds(304 * NS, 16)],
                            out_h.at[pl.ds(HALF0 + 304 * NS, 16)])

    return k(m3, row2)


def kernel(h, edge_index, coord_diff, W_in, b_in, W_out, b_out,
           We1, be1, We2, be2, Wn1, bn1, Wn2, bn2):
    row = edge_index[0].astype(jnp.int32)
    col = edge_index[1].astype(jnp.int32)
    pad = ((0, NCHUNK_PAD - NCHUNK), (0, 0))
    row2g = jnp.pad(row.reshape(NCHUNK, CHUNK), pad)
    col2g = jnp.pad(col.reshape(NCHUNK, CHUNK), pad)
    row2s = jnp.pad(row.reshape(NCHUNK, CHUNK), pad, constant_values=N)
    cdp = jnp.pad(coord_diff, ((0, NCHUNK_PAD * CHUNK - E), (0, 0)))
    rowH = (row2g[:HCH], row2g[HCH:])
    colH = (col2g[:HCH], col2g[HCH:])
    rowHs = (row2s[:HCH], row2s[HCH:])
    cdH = (cdp[:EH], cdp[EH:])
    b_in_r = b_in.reshape(1, D)
    b_out_r = b_out.reshape(1, D)

    x, xa, xb = _tc_prologue(
        h, W_in, b_in_r, We1[0, :D, :], be1[0].reshape(1, D), We1[0, D:2 * D, :])

    out = None
    for l in range(L):
        w_r = We1[l, 2 * D, :].reshape(1, D)
        be2_r = be2[l].reshape(1, D)
        srcA0, dstB0 = _sc_gather(xa, xb, rowH[0], colH[0])
        m0 = _tc_edge(srcA0.reshape(EH, D), dstB0.reshape(EH, D), cdH[0],
                      w_r, We2[l], be2_r)
        srcA1, dstB1 = _sc_gather(xa, xb, rowH[1], colH[1])
        a0 = _sc_scatter(m0.reshape(HCH, CHUNK, D), rowHs[0], HCH)
        m1 = _tc_edge(srcA1.reshape(EH, D), dstB1.reshape(EH, D), cdH[1],
                      w_r, We2[l], be2_r)
        a1 = _sc_scatter(m1.reshape(HCH, CHUNK, D), rowHs[1], NCHUNK - HCH)
        if l < L - 1:
            x, xa, xb = _tc_node(
                x, a0, a1, Wn1[l, :D, :], Wn1[l, D:, :], bn1[l].reshape(1, D),
                Wn2[l], bn2[l].reshape(1, D),
                We1[l + 1, :D, :], be1[l + 1].reshape(1, D), We1[l + 1, D:2 * D, :])
        else:
            out = _tc_node_last(
                x, a0, a1, Wn1[l, :D, :], Wn1[l, D:, :], bn1[l].reshape(1, D),
                Wn2[l], bn2[l].reshape(1, D), W_out, b_out_r)
    return out


# speedup vs baseline: 1.5206x; 1.5206x over previous
import functools

import jax
import jax.numpy as jnp
from jax import lax
from jax.experimental import pallas as pl
from jax.experimental.pallas import tpu as pltpu
from jax.experimental.pallas import tpu_sc as plsc

N = 10000
E = 320000
D = 128
L = 4

NC = 2
NS = 16
NW = NC * NS
CHUNK = 128
NCHUNK = E // CHUNK
CPW = 80
NCHUNK_PAD = CPW * NW
HCH = NCHUNK_PAD // 2
EH = HCH * CHUNK
CPWH = HCH // NW
CPTH = HCH // NS
RPT = 624
TAIL = N - RPT * NS
ZR = 208

def _sc_mesh():
    return plsc.VectorSubcoreMesh(
        core_axis_name="c", subcore_axis_name="s", num_cores=NC, num_subcores=NS)

BN = 2000
BE = 2560


def _silu(v):
    return v * jax.nn.sigmoid(v)


def _full_spec(shape):
    return pl.BlockSpec(shape, lambda i: tuple(0 for _ in shape))


def _tc_prologue(h, W_in, b_in, A0, be10, B0):
    def body(h_ref, win_ref, bin_ref, a_ref, be_ref, b_ref, x_ref, xa_ref, xb_ref):
        x = jnp.dot(h_ref[...], win_ref[...], preferred_element_type=jnp.float32)
        x = x + bin_ref[...]
        x_ref[...] = x
        xa_ref[...] = jnp.dot(x, a_ref[...], preferred_element_type=jnp.float32) + be_ref[...]
        xb_ref[...] = jnp.dot(x, b_ref[...], preferred_element_type=jnp.float32)

    grid = (N // BN,)
    blk = pl.BlockSpec((BN, D), lambda i: (i, 0))
    return pl.pallas_call(
        body,
        grid=grid,
        in_specs=[blk, _full_spec((D, D)), _full_spec((1, D)),
                  _full_spec((D, D)), _full_spec((1, D)), _full_spec((D, D))],
        out_specs=[blk, blk, blk],
        out_shape=[jax.ShapeDtypeStruct((N, D), jnp.float32)] * 3,
    )(h, W_in, b_in, A0, be10, B0)


def _tc_edge(srcA, dstB, cd, w_r, We2, be2):
    def body(sa_ref, db_ref, cd_ref, wr_ref, w2_ref, b2_ref, m_ref):
        c = cd_ref[...]
        radial = jnp.sum(c * c, axis=1, keepdims=True)
        pre = sa_ref[...] + db_ref[...] + radial * wr_ref[...]
        m1 = _silu(pre)
        m_ref[...] = _silu(
            jnp.dot(m1, w2_ref[...], preferred_element_type=jnp.float32) + b2_ref[...])

    grid = (EH // BE,)
    blk = pl.BlockSpec((BE, D), lambda i: (i, 0))
    return pl.pallas_call(
        body,
        grid=grid,
        in_specs=[blk, blk, pl.BlockSpec((BE, 3), lambda i: (i, 0)),
                  _full_spec((1, D)), _full_spec((D, D)), _full_spec((1, D))],
        out_specs=blk,
        out_shape=jax.ShapeDtypeStruct((EH, D), jnp.float32),
    )(srcA, dstB, cd, w_r, We2, be2)


def _tc_node(x, a0, a1, Wn1x, Wn1a, bn1, Wn2, bn2, An, ben, Bn):
    def body(x_ref, a0_ref, a1_ref, w1x_ref, w1a_ref, b1_ref, w2_ref, b2_ref,
             an_ref, ben_ref, bn_ref, xn_ref, xa_ref, xb_ref):
        x = x_ref[...]
        agg = a0_ref[...] + a1_ref[...]
        hmid = _silu(
            jnp.dot(x, w1x_ref[...], preferred_element_type=jnp.float32)
            + jnp.dot(agg, w1a_ref[...], preferred_element_type=jnp.float32)
            + b1_ref[...])
        xn = jnp.dot(hmid, w2_ref[...], preferred_element_type=jnp.float32) + b2_ref[...]
        xn_ref[...] = xn
        xa_ref[...] = jnp.dot(xn, an_ref[...], preferred_element_type=jnp.float32) + ben_ref[...]
        xb_ref[...] = jnp.dot(xn, bn_ref[...], preferred_element_type=jnp.float32)

    grid = (N // BN,)
    blk = pl.BlockSpec((BN, D), lambda i: (i, 0))
    return pl.pallas_call(
        body,
        grid=grid,
        in_specs=[blk, blk, blk,
                  _full_spec((D, D)), _full_spec((D, D)), _full_spec((1, D)),
                  _full_spec((D, D)), _full_spec((1, D)),
                  _full_spec((D, D)), _full_spec((1, D)), _full_spec((D, D))],
        out_specs=[blk, blk, blk],
        out_shape=[jax.ShapeDtypeStruct((N, D), jnp.float32)] * 3,
    )(x, a0, a1, Wn1x, Wn1a, bn1, Wn2, bn2, An, ben, Bn)


def _tc_node_last(x, a0, a1, Wn1x, Wn1a, bn1, Wn2, bn2, W_out, b_out):
    def body(x_ref, a0_ref, a1_ref, w1x_ref, w1a_ref, b1_ref, w2_ref, b2_ref,
             wo_ref, bo_ref, o_ref):
        x = x_ref[...]
        agg = a0_ref[...] + a1_ref[...]
        hmid = _silu(
            jnp.dot(x, w1x_ref[...], preferred_element_type=jnp.float32)
            + jnp.dot(agg, w1a_ref[...], preferred_element_type=jnp.float32)
            + b1_ref[...])
        xn = jnp.dot(hmid, w2_ref[...], preferred_element_type=jnp.float32) + b2_ref[...]
        o_ref[...] = jnp.dot(xn, wo_ref[...], preferred_element_type=jnp.float32) + bo_ref[...]

    grid = (N // BN,)
    blk = pl.BlockSpec((BN, D), lambda i: (i, 0))
    return pl.pallas_call(
        body,
        grid=grid,
        in_specs=[blk, blk, blk,
                  _full_spec((D, D)), _full_spec((D, D)), _full_spec((1, D)),
                  _full_spec((D, D)), _full_spec((1, D)),
                  _full_spec((D, D)), _full_spec((1, D))],
        out_specs=blk,
        out_shape=jax.ShapeDtypeStruct((N, D), jnp.float32),
    )(x, a0, a1, Wn1x, Wn1a, bn1, Wn2, bn2, W_out, b_out)


def _sc_gather(xa, xb, row2, col2):
    @functools.partial(
        pl.kernel,
        out_type=(jax.ShapeDtypeStruct((HCH, CHUNK, D), jnp.float32),
                  jax.ShapeDtypeStruct((HCH, CHUNK, D), jnp.float32)),
        mesh=_sc_mesh(),
        scratch_types=[
            pltpu.VMEM((CPWH, CHUNK), jnp.int32),
            pltpu.VMEM((CPWH, CHUNK), jnp.int32),
            pltpu.VMEM((2, CHUNK, D), jnp.float32),
            pltpu.VMEM((2, CHUNK, D), jnp.float32),
            [pltpu.SemaphoreType.DMA] * 2,
            [pltpu.SemaphoreType.DMA] * 2,
            [pltpu.SemaphoreType.DMA] * 2,
            [pltpu.SemaphoreType.DMA] * 2,
        ],
    )
    def k(xa_h, xb_h, row_h, col_h, srcA_h, dstB_h, idx_r, idx_c,
          bufA, bufB, sgA, sgB, swA, swB):
        w = lax.axis_index("s") * NC + lax.axis_index("c")
        c0 = CPWH * w
        pltpu.sync_copy(row_h.at[pl.ds(c0, CPWH)], idx_r)
        pltpu.sync_copy(col_h.at[pl.ds(c0, CPWH)], idx_c)

        def gath(j, b):
            pltpu.async_copy(xa_h.at[idx_r.at[j]], bufA.at[b], sgA[b])
            pltpu.async_copy(xb_h.at[idx_c.at[j]], bufB.at[b], sgB[b])

        def wait_g(j, b):
            pltpu.make_async_copy(xa_h.at[idx_r.at[j]], bufA.at[b], sgA[b]).wait()
            pltpu.make_async_copy(xb_h.at[idx_c.at[j]], bufB.at[b], sgB[b]).wait()

        def wrb(j, b):
            pltpu.async_copy(bufA.at[b], srcA_h.at[c0 + j], swA[b])
            pltpu.async_copy(bufB.at[b], dstB_h.at[c0 + j], swB[b])

        def wait_w(j, b):
            pltpu.make_async_copy(bufA.at[b], srcA_h.at[c0 + j], swA[b]).wait()
            pltpu.make_async_copy(bufB.at[b], dstB_h.at[c0 + j], swB[b]).wait()

        gath(0, 0)
        gath(1, 1)
        wait_g(0, 0)
        wrb(0, 0)

        def body(p, carry):
            for b2 in range(2):
                kk = 2 * p + b2 + 1
                sb = (b2 + 1) % 2
                so = 1 - sb
                wait_g(kk, sb)
                wrb(kk, sb)
                wait_w(kk - 1, so)

                @pl.when(kk + 1 < CPWH)
                def _():
                    gath(kk + 1, so)
            return carry

        lax.fori_loop(0, (CPWH - 1) // 2, body, 0)
        kk = CPWH - 1
        wait_g(kk, 1)
        wrb(kk, 1)
        wait_w(kk - 1, 0)
        wait_w(kk, 1)

    return k(xa, xb, row2, col2)


HALF0 = 5120
ACC_ROWS = 5136
DUMP = 5128
CPT = 160


def _sc_scatter(m3, row2, nreal):
    @functools.partial(
        pl.kernel,
        out_type=jax.ShapeDtypeStruct((N, D), jnp.float32),
        mesh=_sc_mesh(),
        scratch_types=[
            pltpu.VMEM((CPTH, CHUNK), jnp.int32),
            pltpu.VMEM((CPTH, CHUNK), jnp.int32),
            pltpu.VMEM((2, CHUNK, D), jnp.float32),
            pltpu.VMEM((16, D), jnp.float32),
            pltpu.VMEM_SHARED((ACC_ROWS, D), jnp.float32),
            [pltpu.SemaphoreType.DMA] * 2,
            [pltpu.SemaphoreType.DMA] * 2,
        ],
    )
    def k(m_h, row_h, out_h, idx, idx2, buf, zbuf, acc_sh, sg, sw):
        c = lax.axis_index("c")
        s = lax.axis_index("s")
        base = c * HALF0
        owned = jnp.where(c == 0, HALF0, N - HALF0)
        j0 = CPTH * s
        nj = jnp.clip(nreal - j0, 0, CPTH)
        pltpu.sync_copy(row_h.at[pl.ds(j0, CPTH)], idx)

        zv = jnp.zeros((16,), jnp.float32)
        for r in range(16):
            for g in range(D // 16):
                zbuf[r, g * 16:(g + 1) * 16] = zv

        def remap(j, carry):
            for g in range(D // 16):
                v = idx[j, g * 16:(g + 1) * 16] - base
                ok = (v >= 0) & (v < owned)
                idx2[j, g * 16:(g + 1) * 16] = jnp.where(ok, v, DUMP)
            return carry

        lax.fori_loop(0, nj, remap, 0)

        tb = pl.multiple_of(s * jnp.where(c == 0, HALF0 // NS, 304), 16)
        n16 = jnp.where(c == 0, (HALF0 // NS) // 16, 304 // 16)

        def zero(t, carry):
            off = pl.multiple_of(tb + 16 * t, 16)
            pltpu.sync_copy(zbuf, acc_sh.at[pl.ds(off, 16)])
            return carry

        lax.fori_loop(0, n16, zero, 0)

        @pl.when((c == 1) & (s == NS - 1))
        def _ztail():
            pltpu.sync_copy(zbuf, acc_sh.at[pl.ds(304 * NS, 16)])

        plsc.subcore_barrier()

        def load(j, b):
            pltpu.async_copy(m_h.at[j0 + j], buf.at[b], sg[b])

        def wait_l(j, b):
            pltpu.make_async_copy(m_h.at[j0 + j], buf.at[b], sg[b]).wait()

        def scat(j, b):
            pltpu.async_copy(buf.at[b], acc_sh.at[idx2.at[j]], sw[b], add=True)

        def wait_s(j, b):
            pltpu.make_async_copy(buf.at[b], acc_sh.at[idx2.at[j]], sw[b]).wait()

        load(0, 0)
        load(1, 1)
        wait_l(0, 0)
        scat(0, 0)

        def body(p, carry):
            for b2 in range(2):
                kk = 2 * p + b2 + 1
                sb = (b2 + 1) % 2
                so = 1 - sb
                wait_l(kk, sb)
                scat(kk, sb)
                wait_s(kk - 1, so)

                @pl.when(kk + 1 < nj)
                def _():
                    load(kk + 1, so)
            return carry

        lax.fori_loop(0, (nj - 1) // 2, body, 0)
        kk = nj - 1
        wait_l(kk, 1)
        scat(kk, 1)
        wait_s(kk - 1, 0)
        wait_s(kk, 1)
        plsc.subcore_barrier()

        @pl.when(c == 0)
        def _wb0():
            off = pl.multiple_of((HALF0 // NS) * s, 8)
            pltpu.sync_copy(acc_sh.at[pl.ds(off, HALF0 // NS)],
                            out_h.at[pl.ds(off, HALF0 // NS)])

        @pl.when(c == 1)
        def _wb1():
            off = pl.multiple_of(304 * s, 8)
            pltpu.sync_copy(acc_sh.at[pl.ds(off, 304)],
                            out_h.at[pl.ds(HALF0 + off, 304)])

        @pl.when((c == 1) & (s == NS - 1))
        def _wb1tail():
            pltpu.sync_copy(acc_sh.at[pl.ds(304 * NS, 16)],
                            out_h.at[pl.ds(HALF0 + 304 * NS, 16)])

    return k(m3, row2)


def kernel(h, edge_index, coord_diff, W_in, b_in, W_out, b_out,
           We1, be1, We2, be2, Wn1, bn1, Wn2, bn2):
    row = edge_index[0].astype(jnp.int32)
    col = edge_index[1].astype(jnp.int32)
    pad = ((0, NCHUNK_PAD - NCHUNK), (0, 0))
    spread = (jnp.arange((NCHUNK_PAD - NCHUNK) * CHUNK, dtype=jnp.int32)
              .reshape(NCHUNK_PAD - NCHUNK, CHUNK) * 79) % N
    row2g = jnp.concatenate([row.reshape(NCHUNK, CHUNK), spread])
    col2g = jnp.concatenate([col.reshape(NCHUNK, CHUNK), spread])
    row2s = jnp.pad(row.reshape(NCHUNK, CHUNK), pad, constant_values=N)
    cdp = jnp.pad(coord_diff, ((0, NCHUNK_PAD * CHUNK - E), (0, 0)))
    rowH = (row2g[:HCH], row2g[HCH:])
    colH = (col2g[:HCH], col2g[HCH:])
    rowHs = (row2s[:HCH], row2s[HCH:])
    cdH = (cdp[:EH], cdp[EH:])
    b_in_r = b_in.reshape(1, D)
    b_out_r = b_out.reshape(1, D)

    x, xa, xb = _tc_prologue(
        h, W_in, b_in_r, We1[0, :D, :], be1[0].reshape(1, D), We1[0, D:2 * D, :])

    out = None
    for l in range(L):
        w_r = We1[l, 2 * D, :].reshape(1, D)
        be2_r = be2[l].reshape(1, D)
        srcA0, dstB0 = _sc_gather(xa, xb, rowH[0], colH[0])
        m0 = _tc_edge(srcA0.reshape(EH, D), dstB0.reshape(EH, D), cdH[0],
                      w_r, We2[l], be2_r)
        srcA1, dstB1 = _sc_gather(xa, xb, rowH[1], colH[1])
        a0 = _sc_scatter(m0.reshape(HCH, CHUNK, D), rowHs[0], HCH)
        m1 = _tc_edge(srcA1.reshape(EH, D), dstB1.reshape(EH, D), cdH[1],
                      w_r, We2[l], be2_r)
        a1 = _sc_scatter(m1.reshape(HCH, CHUNK, D), rowHs[1], NCHUNK - HCH)
        if l < L - 1:
            x, xa, xb = _tc_node(
                x, a0, a1, Wn1[l, :D, :], Wn1[l, D:, :], bn1[l].reshape(1, D),
                Wn2[l], bn2[l].reshape(1, D),
                We1[l + 1, :D, :], be1[l + 1].reshape(1, D), We1[l + 1, D:2 * D, :])
        else:
            out = _tc_node_last(
                x, a0, a1, Wn1[l, :D, :], Wn1[l, D:, :], bn1[l].reshape(1, D),
                Wn2[l], bn2[l].reshape(1, D), W_out, b_out_r)
    return out


# speedup vs baseline: 1.6121x; 1.0602x over previous
import functools

import jax
import jax.numpy as jnp
from jax import lax
from jax.experimental import pallas as pl
from jax.experimental.pallas import tpu as pltpu
from jax.experimental.pallas import tpu_sc as plsc

N = 10000
E = 320000
D = 128
L = 4

NC = 2
NS = 16
NW = NC * NS
CHUNK = 128
NCHUNK = E // CHUNK
CPW = 80
NCHUNK_PAD = CPW * NW
HCH = NCHUNK_PAD // 2
NCH1 = NCHUNK - HCH
EH = HCH * CHUNK
CPWH = HCH // NW
CPTH = HCH // NS
RPT = 624
TAIL = N - RPT * NS
ZR = 208

def _sc_mesh():
    return plsc.VectorSubcoreMesh(
        core_axis_name="c", subcore_axis_name="s", num_cores=NC, num_subcores=NS)

BN = 2000
BE = 2560


def _silu(v):
    return v * jax.nn.sigmoid(v)


def _full_spec(shape):
    return pl.BlockSpec(shape, lambda i: tuple(0 for _ in shape))


def _tc_prologue(h, W_in, b_in, A0, be10, B0):
    def body(h_ref, win_ref, bin_ref, a_ref, be_ref, b_ref, x_ref, xa_ref, xb_ref):
        x = jnp.dot(h_ref[...], win_ref[...], preferred_element_type=jnp.float32)
        x = x + bin_ref[...]
        x_ref[...] = x
        xa_ref[...] = jnp.dot(x, a_ref[...], preferred_element_type=jnp.float32) + be_ref[...]
        xb_ref[...] = jnp.dot(x, b_ref[...], preferred_element_type=jnp.float32)

    grid = (N // BN,)
    blk = pl.BlockSpec((BN, D), lambda i: (i, 0))
    return pl.pallas_call(
        body,
        grid=grid,
        in_specs=[blk, _full_spec((D, D)), _full_spec((1, D)),
                  _full_spec((D, D)), _full_spec((1, D)), _full_spec((D, D))],
        out_specs=[blk, blk, blk],
        out_shape=[jax.ShapeDtypeStruct((N, D), jnp.float32)] * 3,
    )(h, W_in, b_in, A0, be10, B0)


def _tc_edge(srcA, dstB, cd, w_r, We2, be2, ne):
    def body(sa_ref, db_ref, cd_ref, wr_ref, w2_ref, b2_ref, m_ref):
        c = cd_ref[...]
        radial = jnp.sum(c * c, axis=1, keepdims=True)
        pre = sa_ref[...] + db_ref[...] + radial * wr_ref[...]
        m1 = _silu(pre)
        m_ref[...] = _silu(
            jnp.dot(m1, w2_ref[...], preferred_element_type=jnp.float32) + b2_ref[...])

    grid = (ne // BE,)
    blk = pl.BlockSpec((BE, D), lambda i: (i, 0))
    return pl.pallas_call(
        body,
        grid=grid,
        in_specs=[blk, blk, pl.BlockSpec((BE, 3), lambda i: (i, 0)),
                  _full_spec((1, D)), _full_spec((D, D)), _full_spec((1, D))],
        out_specs=blk,
        out_shape=jax.ShapeDtypeStruct((ne, D), jnp.float32),
    )(srcA, dstB, cd, w_r, We2, be2)


def _tc_node(x, a0, a1, Wn1x, Wn1a, bn1, Wn2, bn2, An, ben, Bn):
    def body(x_ref, a0_ref, a1_ref, w1x_ref, w1a_ref, b1_ref, w2_ref, b2_ref,
             an_ref, ben_ref, bn_ref, xn_ref, xa_ref, xb_ref):
        x = x_ref[...]
        agg = a0_ref[...] + a1_ref[...]
        hmid = _silu(
            jnp.dot(x, w1x_ref[...], preferred_element_type=jnp.float32)
            + jnp.dot(agg, w1a_ref[...], preferred_element_type=jnp.float32)
            + b1_ref[...])
        xn = jnp.dot(hmid, w2_ref[...], preferred_element_type=jnp.float32) + b2_ref[...]
        xn_ref[...] = xn
        xa_ref[...] = jnp.dot(xn, an_ref[...], preferred_element_type=jnp.float32) + ben_ref[...]
        xb_ref[...] = jnp.dot(xn, bn_ref[...], preferred_element_type=jnp.float32)

    grid = (N // BN,)
    blk = pl.BlockSpec((BN, D), lambda i: (i, 0))
    return pl.pallas_call(
        body,
        grid=grid,
        in_specs=[blk, blk, blk,
                  _full_spec((D, D)), _full_spec((D, D)), _full_spec((1, D)),
                  _full_spec((D, D)), _full_spec((1, D)),
                  _full_spec((D, D)), _full_spec((1, D)), _full_spec((D, D))],
        out_specs=[blk, blk, blk],
        out_shape=[jax.ShapeDtypeStruct((N, D), jnp.float32)] * 3,
    )(x, a0, a1, Wn1x, Wn1a, bn1, Wn2, bn2, An, ben, Bn)


def _tc_node_last(x, a0, a1, Wn1x, Wn1a, bn1, Wn2, bn2, W_out, b_out):
    def body(x_ref, a0_ref, a1_ref, w1x_ref, w1a_ref, b1_ref, w2_ref, b2_ref,
             wo_ref, bo_ref, o_ref):
        x = x_ref[...]
        agg = a0_ref[...] + a1_ref[...]
        hmid = _silu(
            jnp.dot(x, w1x_ref[...], preferred_element_type=jnp.float32)
            + jnp.dot(agg, w1a_ref[...], preferred_element_type=jnp.float32)
            + b1_ref[...])
        xn = jnp.dot(hmid, w2_ref[...], preferred_element_type=jnp.float32) + b2_ref[...]
        o_ref[...] = jnp.dot(xn, wo_ref[...], preferred_element_type=jnp.float32) + bo_ref[...]

    grid = (N // BN,)
    blk = pl.BlockSpec((BN, D), lambda i: (i, 0))
    return pl.pallas_call(
        body,
        grid=grid,
        in_specs=[blk, blk, blk,
                  _full_spec((D, D)), _full_spec((D, D)), _full_spec((1, D)),
                  _full_spec((D, D)), _full_spec((1, D)),
                  _full_spec((D, D)), _full_spec((1, D))],
        out_specs=blk,
        out_shape=jax.ShapeDtypeStruct((N, D), jnp.float32),
    )(x, a0, a1, Wn1x, Wn1a, bn1, Wn2, bn2, W_out, b_out)


def _sc_gather(xa, xb, row2, col2, nreal):
    @functools.partial(
        pl.kernel,
        out_type=(jax.ShapeDtypeStruct((nreal, CHUNK, D), jnp.float32),
                  jax.ShapeDtypeStruct((nreal, CHUNK, D), jnp.float32)),
        mesh=_sc_mesh(),
        scratch_types=[
            pltpu.VMEM((CPWH, CHUNK), jnp.int32),
            pltpu.VMEM((CPWH, CHUNK), jnp.int32),
            pltpu.VMEM((2, CHUNK, D), jnp.float32),
            pltpu.VMEM((2, CHUNK, D), jnp.float32),
            [pltpu.SemaphoreType.DMA] * 2,
            [pltpu.SemaphoreType.DMA] * 2,
            [pltpu.SemaphoreType.DMA] * 2,
            [pltpu.SemaphoreType.DMA] * 2,
        ],
    )
    def k(xa_h, xb_h, row_h, col_h, srcA_h, dstB_h, idx_r, idx_c,
          bufA, bufB, sgA, sgB, swA, swB):
        w = lax.axis_index("s") * NC + lax.axis_index("c")
        c0 = CPWH * w
        nj = jnp.clip(nreal - c0, 0, CPWH)
        pltpu.sync_copy(row_h.at[pl.ds(c0, CPWH)], idx_r)
        pltpu.sync_copy(col_h.at[pl.ds(c0, CPWH)], idx_c)

        def gath(j, b):
            pltpu.async_copy(xa_h.at[idx_r.at[j]], bufA.at[b], sgA[b])
            pltpu.async_copy(xb_h.at[idx_c.at[j]], bufB.at[b], sgB[b])

        def wait_g(j, b):
            pltpu.make_async_copy(xa_h.at[idx_r.at[j]], bufA.at[b], sgA[b]).wait()
            pltpu.make_async_copy(xb_h.at[idx_c.at[j]], bufB.at[b], sgB[b]).wait()

        def wrb(j, b):
            pltpu.async_copy(bufA.at[b], srcA_h.at[c0 + j], swA[b])
            pltpu.async_copy(bufB.at[b], dstB_h.at[c0 + j], swB[b])

        def wait_w(j, b):
            pltpu.make_async_copy(bufA.at[b], srcA_h.at[c0 + j], swA[b]).wait()
            pltpu.make_async_copy(bufB.at[b], dstB_h.at[c0 + j], swB[b]).wait()

        @pl.when(nj > 0)
        def _pipeline():
            gath(0, 0)
            gath(1, 1)
            wait_g(0, 0)
            wrb(0, 0)

            def body(p, carry):
                for b2 in range(2):
                    kk = 2 * p + b2 + 1
                    sb = (b2 + 1) % 2
                    so = 1 - sb
                    wait_g(kk, sb)
                    wrb(kk, sb)
                    wait_w(kk - 1, so)

                    @pl.when(kk + 1 < nj)
                    def _():
                        gath(kk + 1, so)
                return carry

            lax.fori_loop(0, (nj - 1) // 2, body, 0)
            kk = nj - 1
            wait_g(kk, 1)
            wrb(kk, 1)
            wait_w(kk - 1, 0)
            wait_w(kk, 1)

    return k(xa, xb, row2, col2)


HALF0 = 5120
ACC_ROWS = 5136
DUMP = 5128
CPT = 160


def _sc_scatter(m3, row2, nreal):
    @functools.partial(
        pl.kernel,
        out_type=jax.ShapeDtypeStruct((N, D), jnp.float32),
        mesh=_sc_mesh(),
        scratch_types=[
            pltpu.VMEM((CPTH, CHUNK), jnp.int32),
            pltpu.VMEM((CPTH, CHUNK), jnp.int32),
            pltpu.VMEM((2, CHUNK, D), jnp.float32),
            pltpu.VMEM((16, D), jnp.float32),
            pltpu.VMEM_SHARED((ACC_ROWS, D), jnp.float32),
            [pltpu.SemaphoreType.DMA] * 2,
            [pltpu.SemaphoreType.DMA] * 2,
        ],
    )
    def k(m_h, row_h, out_h, idx, idx2, buf, zbuf, acc_sh, sg, sw):
        c = lax.axis_index("c")
        s = lax.axis_index("s")
        base = c * HALF0
        owned = jnp.where(c == 0, HALF0, N - HALF0)
        j0 = CPTH * s
        nj = jnp.clip(nreal - j0, 0, CPTH)
        pltpu.sync_copy(row_h.at[pl.ds(j0, CPTH)], idx)

        zv = jnp.zeros((16,), jnp.float32)
        for r in range(16):
            for g in range(D // 16):
                zbuf[r, g * 16:(g + 1) * 16] = zv

        def remap(j, carry):
            for g in range(D // 16):
                v = idx[j, g * 16:(g + 1) * 16] - base
                ok = (v >= 0) & (v < owned)
                idx2[j, g * 16:(g + 1) * 16] = jnp.where(ok, v, DUMP)
            return carry

        lax.fori_loop(0, nj, remap, 0)

        tb = pl.multiple_of(s * jnp.where(c == 0, HALF0 // NS, 304), 16)
        n16 = jnp.where(c == 0, (HALF0 // NS) // 16, 304 // 16)

        def zero(t, carry):
            off = pl.multiple_of(tb + 16 * t, 16)
            pltpu.sync_copy(zbuf, acc_sh.at[pl.ds(off, 16)])
            return carry

        lax.fori_loop(0, n16, zero, 0)

        @pl.when((c == 1) & (s == NS - 1))
        def _ztail():
            pltpu.sync_copy(zbuf, acc_sh.at[pl.ds(304 * NS, 16)])

        plsc.subcore_barrier()

        def load(j, b):
            pltpu.async_copy(m_h.at[j0 + j], buf.at[b], sg[b])

        def wait_l(j, b):
            pltpu.make_async_copy(m_h.at[j0 + j], buf.at[b], sg[b]).wait()

        def scat(j, b):
            pltpu.async_copy(buf.at[b], acc_sh.at[idx2.at[j]], sw[b], add=True)

        def wait_s(j, b):
            pltpu.make_async_copy(buf.at[b], acc_sh.at[idx2.at[j]], sw[b]).wait()

        load(0, 0)
        load(1, 1)
        wait_l(0, 0)
        scat(0, 0)

        def body(p, carry):
            for b2 in range(2):
                kk = 2 * p + b2 + 1
                sb = (b2 + 1) % 2
                so = 1 - sb
                wait_l(kk, sb)
                scat(kk, sb)
                wait_s(kk - 1, so)

                @pl.when(kk + 1 < nj)
                def _():
                    load(kk + 1, so)
            return carry

        lax.fori_loop(0, (nj - 1) // 2, body, 0)
        kk = nj - 1
        wait_l(kk, 1)
        scat(kk, 1)
        wait_s(kk - 1, 0)
        wait_s(kk, 1)
        plsc.subcore_barrier()

        @pl.when(c == 0)
        def _wb0():
            off = pl.multiple_of((HALF0 // NS) * s, 8)
            pltpu.sync_copy(acc_sh.at[pl.ds(off, HALF0 // NS)],
                            out_h.at[pl.ds(off, HALF0 // NS)])

        @pl.when(c == 1)
        def _wb1():
            off = pl.multiple_of(304 * s, 8)
            pltpu.sync_copy(acc_sh.at[pl.ds(off, 304)],
                            out_h.at[pl.ds(HALF0 + off, 304)])

        @pl.when((c == 1) & (s == NS - 1))
        def _wb1tail():
            pltpu.sync_copy(acc_sh.at[pl.ds(304 * NS, 16)],
                            out_h.at[pl.ds(HALF0 + 304 * NS, 16)])

    return k(m3, row2)


def kernel(h, edge_index, coord_diff, W_in, b_in, W_out, b_out,
           We1, be1, We2, be2, Wn1, bn1, Wn2, bn2):
    row = edge_index[0].astype(jnp.int32)
    col = edge_index[1].astype(jnp.int32)
    pad = ((0, NCHUNK_PAD - NCHUNK), (0, 0))
    spread = (jnp.arange((NCHUNK_PAD - NCHUNK) * CHUNK, dtype=jnp.int32)
              .reshape(NCHUNK_PAD - NCHUNK, CHUNK) * 79) % N
    row2g = jnp.concatenate([row.reshape(NCHUNK, CHUNK), spread])
    col2g = jnp.concatenate([col.reshape(NCHUNK, CHUNK), spread])
    row2s = jnp.pad(row.reshape(NCHUNK, CHUNK), pad, constant_values=N)

    rowH = (row2g[:HCH], row2g[HCH:])
    colH = (col2g[:HCH], col2g[HCH:])
    rowHs = (row2s[:HCH], row2s[HCH:])
    cdH = (coord_diff[:EH], coord_diff[EH:])
    b_in_r = b_in.reshape(1, D)
    b_out_r = b_out.reshape(1, D)

    x, xa, xb = _tc_prologue(
        h, W_in, b_in_r, We1[0, :D, :], be1[0].reshape(1, D), We1[0, D:2 * D, :])

    out = None
    for l in range(L):
        w_r = We1[l, 2 * D, :].reshape(1, D)
        be2_r = be2[l].reshape(1, D)
        srcA0, dstB0 = _sc_gather(xa, xb, rowH[0], colH[0], HCH)
        m0 = _tc_edge(srcA0.reshape(EH, D), dstB0.reshape(EH, D), cdH[0],
                      w_r, We2[l], be2_r, EH)
        srcA1, dstB1 = _sc_gather(xa, xb, rowH[1], colH[1], NCH1)
        a0 = _sc_scatter(m0.reshape(HCH, CHUNK, D), rowHs[0], HCH)
        m1 = _tc_edge(srcA1.reshape(NCH1 * CHUNK, D), dstB1.reshape(NCH1 * CHUNK, D),
                      cdH[1], w_r, We2[l], be2_r, NCH1 * CHUNK)
        a1 = _sc_scatter(m1.reshape(NCH1, CHUNK, D), rowHs[1], NCH1)
        if l < L - 1:
            x, xa, xb = _tc_node(
                x, a0, a1, Wn1[l, :D, :], Wn1[l, D:, :], bn1[l].reshape(1, D),
                Wn2[l], bn2[l].reshape(1, D),
                We1[l + 1, :D, :], be1[l + 1].reshape(1, D), We1[l + 1, D:2 * D, :])
        else:
            out = _tc_node_last(
                x, a0, a1, Wn1[l, :D, :], Wn1[l, D:, :], bn1[l].reshape(1, D),
                Wn2[l], bn2[l].reshape(1, D), W_out, b_out_r)
    return out


# speedup vs baseline: 1.6140x; 1.0012x over previous
import functools

import jax
import jax.numpy as jnp
from jax import lax
from jax.experimental import pallas as pl
from jax.experimental.pallas import tpu as pltpu
from jax.experimental.pallas import tpu_sc as plsc

N = 10000
E = 320000
D = 128
L = 4

NC = 2
NS = 16
NW = NC * NS
CHUNK = 128
NCHUNK = E // CHUNK
CPW = 80
NCHUNK_PAD = CPW * NW
HCH = NCHUNK_PAD // 2
NCH1 = NCHUNK - HCH
EH = HCH * CHUNK
CPWH = HCH // NW
CPTH = HCH // NS
RPT = 624
TAIL = N - RPT * NS
ZR = 208

def _sc_mesh():
    return plsc.VectorSubcoreMesh(
        core_axis_name="c", subcore_axis_name="s", num_cores=NC, num_subcores=NS)

BN = 2000
BE = 2560


def _silu(v):
    return v * jax.nn.sigmoid(v)


def _full_spec(shape):
    return pl.BlockSpec(shape, lambda i: tuple(0 for _ in shape))


def _tc_prologue(h, W_in, b_in, A0, be10, B0):
    def body(h_ref, win_ref, bin_ref, a_ref, be_ref, b_ref, x_ref, xa_ref, xb_ref):
        x = jnp.dot(h_ref[...], win_ref[...], preferred_element_type=jnp.float32)
        x = x + bin_ref[...]
        x_ref[...] = x
        xa_ref[...] = jnp.dot(x, a_ref[...], preferred_element_type=jnp.float32) + be_ref[...]
        xb_ref[...] = jnp.dot(x, b_ref[...], preferred_element_type=jnp.float32)

    grid = (N // BN,)
    blk = pl.BlockSpec((BN, D), lambda i: (i, 0))
    return pl.pallas_call(
        body,
        grid=grid,
        in_specs=[blk, _full_spec((D, D)), _full_spec((1, D)),
                  _full_spec((D, D)), _full_spec((1, D)), _full_spec((D, D))],
        out_specs=[blk, blk, blk],
        out_shape=[jax.ShapeDtypeStruct((N, D), jnp.float32)] * 3,
    )(h, W_in, b_in, A0, be10, B0)


BEC = BE // CHUNK


def _tc_edge(sd, cd, w_r, We2, be2, ne):
    def body(sa_ref, db_ref, cd_ref, wr_ref, w2_ref, b2_ref, m_ref):
        sa = sa_ref[...].reshape(BE, D)
        db = db_ref[...].reshape(BE, D)
        c = cd_ref[...]
        radial = jnp.sum(c * c, axis=1, keepdims=True)
        pre = sa + db + radial * wr_ref[...]
        m1 = _silu(pre)
        m_ref[...] = _silu(
            jnp.dot(m1, w2_ref[...], preferred_element_type=jnp.float32) + b2_ref[...])

    grid = (ne // BE,)
    blk = pl.BlockSpec((BE, D), lambda i: (i, 0))
    return pl.pallas_call(
        body,
        grid=grid,
        in_specs=[pl.BlockSpec((BEC, CHUNK, D), lambda i: (i, 0, 0)),
                  pl.BlockSpec((BEC, CHUNK, D), lambda i: (i, 1, 0)),
                  pl.BlockSpec((BE, 3), lambda i: (i, 0)),
                  _full_spec((1, D)), _full_spec((D, D)), _full_spec((1, D))],
        out_specs=blk,
        out_shape=jax.ShapeDtypeStruct((ne, D), jnp.float32),
    )(sd, sd, cd, w_r, We2, be2)


def _tc_node(x, a0, a1, Wn1x, Wn1a, bn1, Wn2, bn2, An, ben, Bn):
    def body(x_ref, a0_ref, a1_ref, w1x_ref, w1a_ref, b1_ref, w2_ref, b2_ref,
             an_ref, ben_ref, bn_ref, xn_ref, xa_ref, xb_ref):
        x = x_ref[...]
        agg = a0_ref[...] + a1_ref[...]
        hmid = _silu(
            jnp.dot(x, w1x_ref[...], preferred_element_type=jnp.float32)
            + jnp.dot(agg, w1a_ref[...], preferred_element_type=jnp.float32)
            + b1_ref[...])
        xn = jnp.dot(hmid, w2_ref[...], preferred_element_type=jnp.float32) + b2_ref[...]
        xn_ref[...] = xn
        xa_ref[...] = jnp.dot(xn, an_ref[...], preferred_element_type=jnp.float32) + ben_ref[...]
        xb_ref[...] = jnp.dot(xn, bn_ref[...], preferred_element_type=jnp.float32)

    grid = (N // BN,)
    blk = pl.BlockSpec((BN, D), lambda i: (i, 0))
    return pl.pallas_call(
        body,
        grid=grid,
        in_specs=[blk, blk, blk,
                  _full_spec((D, D)), _full_spec((D, D)), _full_spec((1, D)),
                  _full_spec((D, D)), _full_spec((1, D)),
                  _full_spec((D, D)), _full_spec((1, D)), _full_spec((D, D))],
        out_specs=[blk, blk, blk],
        out_shape=[jax.ShapeDtypeStruct((N, D), jnp.float32)] * 3,
    )(x, a0, a1, Wn1x, Wn1a, bn1, Wn2, bn2, An, ben, Bn)


def _tc_node_last(x, a0, a1, Wn1x, Wn1a, bn1, Wn2, bn2, W_out, b_out):
    def body(x_ref, a0_ref, a1_ref, w1x_ref, w1a_ref, b1_ref, w2_ref, b2_ref,
             wo_ref, bo_ref, o_ref):
        x = x_ref[...]
        agg = a0_ref[...] + a1_ref[...]
        hmid = _silu(
            jnp.dot(x, w1x_ref[...], preferred_element_type=jnp.float32)
            + jnp.dot(agg, w1a_ref[...], preferred_element_type=jnp.float32)
            + b1_ref[...])
        xn = jnp.dot(hmid, w2_ref[...], preferred_element_type=jnp.float32) + b2_ref[...]
        o_ref[...] = jnp.dot(xn, wo_ref[...], preferred_element_type=jnp.float32) + bo_ref[...]

    grid = (N // BN,)
    blk = pl.BlockSpec((BN, D), lambda i: (i, 0))
    return pl.pallas_call(
        body,
        grid=grid,
        in_specs=[blk, blk, blk,
                  _full_spec((D, D)), _full_spec((D, D)), _full_spec((1, D)),
                  _full_spec((D, D)), _full_spec((1, D)),
                  _full_spec((D, D)), _full_spec((1, D))],
        out_specs=blk,
        out_shape=jax.ShapeDtypeStruct((N, D), jnp.float32),
    )(x, a0, a1, Wn1x, Wn1a, bn1, Wn2, bn2, W_out, b_out)


def _sc_gather(xa, xb, row2, col2, nreal):
    @functools.partial(
        pl.kernel,
        out_type=jax.ShapeDtypeStruct((nreal, 2 * CHUNK, D), jnp.float32),
        mesh=_sc_mesh(),
        scratch_types=[
            pltpu.VMEM((CPWH, CHUNK), jnp.int32),
            pltpu.VMEM((CPWH, CHUNK), jnp.int32),
            pltpu.VMEM((2, 2 * CHUNK, D), jnp.float32),
            [pltpu.SemaphoreType.DMA] * 2,
            [pltpu.SemaphoreType.DMA] * 2,
            [pltpu.SemaphoreType.DMA] * 2,
        ],
    )
    def k(xa_h, xb_h, row_h, col_h, sd_h, idx_r, idx_c, buf, sgA, sgB, sw):
        w = lax.axis_index("s") * NC + lax.axis_index("c")
        c0 = CPWH * w
        nj = jnp.clip(nreal - c0, 0, CPWH)
        pltpu.sync_copy(row_h.at[pl.ds(c0, CPWH)], idx_r)
        pltpu.sync_copy(col_h.at[pl.ds(c0, CPWH)], idx_c)

        def gath(j, b):
            pltpu.async_copy(xa_h.at[idx_r.at[j]], buf.at[b].at[pl.ds(0, CHUNK)], sgA[b])
            pltpu.async_copy(xb_h.at[idx_c.at[j]], buf.at[b].at[pl.ds(CHUNK, CHUNK)], sgB[b])

        def wait_g(j, b):
            pltpu.make_async_copy(xa_h.at[idx_r.at[j]], buf.at[b].at[pl.ds(0, CHUNK)], sgA[b]).wait()
            pltpu.make_async_copy(xb_h.at[idx_c.at[j]], buf.at[b].at[pl.ds(CHUNK, CHUNK)], sgB[b]).wait()

        def wrb(j, b):
            pltpu.async_copy(buf.at[b], sd_h.at[c0 + j], sw[b])

        def wait_w(j, b):
            pltpu.make_async_copy(buf.at[b], sd_h.at[c0 + j], sw[b]).wait()

        @pl.when(nj > 0)
        def _pipeline():
            gath(0, 0)
            gath(1, 1)
            wait_g(0, 0)
            wrb(0, 0)

            def body(p, carry):
                for b2 in range(2):
                    kk = 2 * p + b2 + 1
                    sb = (b2 + 1) % 2
                    so = 1 - sb
                    wait_g(kk, sb)
                    wrb(kk, sb)
                    wait_w(kk - 1, so)

                    @pl.when(kk + 1 < nj)
                    def _():
                        gath(kk + 1, so)
                return carry

            lax.fori_loop(0, (nj - 1) // 2, body, 0)
            kk = nj - 1
            wait_g(kk, 1)
            wrb(kk, 1)
            wait_w(kk - 1, 0)
            wait_w(kk, 1)

    return k(xa, xb, row2, col2)


HALF0 = 5120
ACC_ROWS = 5136
DUMP = 5128
CPT = 160


def _sc_scatter(m3, row2, nreal):
    @functools.partial(
        pl.kernel,
        out_type=jax.ShapeDtypeStruct((N, D), jnp.float32),
        mesh=_sc_mesh(),
        scratch_types=[
            pltpu.VMEM((CPTH, CHUNK), jnp.int32),
            pltpu.VMEM((CPTH, CHUNK), jnp.int32),
            pltpu.VMEM((2, CHUNK, D), jnp.float32),
            pltpu.VMEM((16, D), jnp.float32),
            pltpu.VMEM_SHARED((ACC_ROWS, D), jnp.float32),
            [pltpu.SemaphoreType.DMA] * 2,
            [pltpu.SemaphoreType.DMA] * 2,
        ],
    )
    def k(m_h, row_h, out_h, idx, idx2, buf, zbuf, acc_sh, sg, sw):
        c = lax.axis_index("c")
        s = lax.axis_index("s")
        base = c * HALF0
        owned = jnp.where(c == 0, HALF0, N - HALF0)
        j0 = CPTH * s
        nj = jnp.clip(nreal - j0, 0, CPTH)
        pltpu.sync_copy(row_h.at[pl.ds(j0, CPTH)], idx)

        zv = jnp.zeros((16,), jnp.float32)
        for r in range(16):
            for g in range(D // 16):
                zbuf[r, g * 16:(g + 1) * 16] = zv

        def remap(j, carry):
            for g in range(D // 16):
                v = idx[j, g * 16:(g + 1) * 16] - base
                ok = (v >= 0) & (v < owned)
                idx2[j, g * 16:(g + 1) * 16] = jnp.where(ok, v, DUMP)
            return carry

        lax.fori_loop(0, nj, remap, 0)

        tb = pl.multiple_of(s * jnp.where(c == 0, HALF0 // NS, 304), 16)
        n16 = jnp.where(c == 0, (HALF0 // NS) // 16, 304 // 16)

        def zero(t, carry):
            off = pl.multiple_of(tb + 16 * t, 16)
            pltpu.sync_copy(zbuf, acc_sh.at[pl.ds(off, 16)])
            return carry

        lax.fori_loop(0, n16, zero, 0)

        @pl.when((c == 1) & (s == NS - 1))
        def _ztail():
            pltpu.sync_copy(zbuf, acc_sh.at[pl.ds(304 * NS, 16)])

        plsc.subcore_barrier()

        def load(j, b):
            pltpu.async_copy(m_h.at[j0 + j], buf.at[b], sg[b])

        def wait_l(j, b):
            pltpu.make_async_copy(m_h.at[j0 + j], buf.at[b], sg[b]).wait()

        def scat(j, b):
            pltpu.async_copy(buf.at[b], acc_sh.at[idx2.at[j]], sw[b], add=True)

        def wait_s(j, b):
            pltpu.make_async_copy(buf.at[b], acc_sh.at[idx2.at[j]], sw[b]).wait()

        load(0, 0)
        load(1, 1)
        wait_l(0, 0)
        scat(0, 0)

        def body(p, carry):
            for b2 in range(2):
                kk = 2 * p + b2 + 1
                sb = (b2 + 1) % 2
                so = 1 - sb
                wait_l(kk, sb)
                scat(kk, sb)
                wait_s(kk - 1, so)

                @pl.when(kk + 1 < nj)
                def _():
                    load(kk + 1, so)
            return carry

        lax.fori_loop(0, (nj - 1) // 2, body, 0)
        kk = nj - 1
        wait_l(kk, 1)
        scat(kk, 1)
        wait_s(kk - 1, 0)
        wait_s(kk, 1)
        plsc.subcore_barrier()

        @pl.when(c == 0)
        def _wb0():
            off = pl.multiple_of((HALF0 // NS) * s, 8)
            pltpu.sync_copy(acc_sh.at[pl.ds(off, HALF0 // NS)],
                            out_h.at[pl.ds(off, HALF0 // NS)])

        @pl.when(c == 1)
        def _wb1():
            off = pl.multiple_of(304 * s, 8)
            pltpu.sync_copy(acc_sh.at[pl.ds(off, 304)],
                            out_h.at[pl.ds(HALF0 + off, 304)])

        @pl.when((c == 1) & (s == NS - 1))
        def _wb1tail():
            pltpu.sync_copy(acc_sh.at[pl.ds(304 * NS, 16)],
                            out_h.at[pl.ds(HALF0 + 304 * NS, 16)])

    return k(m3, row2)


def kernel(h, edge_index, coord_diff, W_in, b_in, W_out, b_out,
           We1, be1, We2, be2, Wn1, bn1, Wn2, bn2):
    row = edge_index[0].astype(jnp.int32)
    col = edge_index[1].astype(jnp.int32)
    pad = ((0, NCHUNK_PAD - NCHUNK), (0, 0))
    spread = (jnp.arange((NCHUNK_PAD - NCHUNK) * CHUNK, dtype=jnp.int32)
              .reshape(NCHUNK_PAD - NCHUNK, CHUNK) * 79) % N
    row2g = jnp.concatenate([row.reshape(NCHUNK, CHUNK), spread])
    col2g = jnp.concatenate([col.reshape(NCHUNK, CHUNK), spread])
    row2s = jnp.pad(row.reshape(NCHUNK, CHUNK), pad, constant_values=N)

    rowH = (row2g[:HCH], row2g[HCH:])
    colH = (col2g[:HCH], col2g[HCH:])
    rowHs = (row2s[:HCH], row2s[HCH:])
    cdH = (coord_diff[:EH], coord_diff[EH:])
    b_in_r = b_in.reshape(1, D)
    b_out_r = b_out.reshape(1, D)

    x, xa, xb = _tc_prologue(
        h, W_in, b_in_r, We1[0, :D, :], be1[0].reshape(1, D), We1[0, D:2 * D, :])

    out = None
    for l in range(L):
        w_r = We1[l, 2 * D, :].reshape(1, D)
        be2_r = be2[l].reshape(1, D)
        sd0 = _sc_gather(xa, xb, rowH[0], colH[0], HCH)
        m0 = _tc_edge(sd0, cdH[0], w_r, We2[l], be2_r, EH)
        sd1 = _sc_gather(xa, xb, rowH[1], colH[1], NCH1)
        a0 = _sc_scatter(m0.reshape(HCH, CHUNK, D), rowHs[0], HCH)
        m1 = _tc_edge(sd1, cdH[1], w_r, We2[l], be2_r, NCH1 * CHUNK)
        a1 = _sc_scatter(m1.reshape(NCH1, CHUNK, D), rowHs[1], NCH1)
        if l < L - 1:
            x, xa, xb = _tc_node(
                x, a0, a1, Wn1[l, :D, :], Wn1[l, D:, :], bn1[l].reshape(1, D),
                Wn2[l], bn2[l].reshape(1, D),
                We1[l + 1, :D, :], be1[l + 1].reshape(1, D), We1[l + 1, D:2 * D, :])
        else:
            out = _tc_node_last(
                x, a0, a1, Wn1[l, :D, :], Wn1[l, D:, :], bn1[l].reshape(1, D),
                Wn2[l], bn2[l].reshape(1, D), W_out, b_out_r)
    return out


# speedup vs baseline: 1.6180x; 1.0025x over previous
import functools

import jax
import jax.numpy as jnp
from jax import lax
from jax.experimental import pallas as pl
from jax.experimental.pallas import tpu as pltpu
from jax.experimental.pallas import tpu_sc as plsc

N = 10000
E = 320000
D = 128
L = 4

NC = 2
NS = 16
NW = NC * NS
CHUNK = 128
NCHUNK = E // CHUNK
CPW = 80
NCHUNK_PAD = CPW * NW
HCH = NCHUNK_PAD // 2
NCH1 = NCHUNK - HCH
EH = HCH * CHUNK
CPWH = HCH // NW
CPTH = HCH // NS
RPT = 624
TAIL = N - RPT * NS
ZR = 208

def _sc_mesh():
    return plsc.VectorSubcoreMesh(
        core_axis_name="c", subcore_axis_name="s", num_cores=NC, num_subcores=NS)

BN = 2000
BE = 2560


def _silu(v):
    return v * jax.nn.sigmoid(v)


def _full_spec(shape):
    return pl.BlockSpec(shape, lambda i: tuple(0 for _ in shape))


def _tc_prologue(h, W_in, b_in, A0, be10, B0):
    def body(h_ref, win_ref, bin_ref, a_ref, be_ref, b_ref, x_ref, xa_ref, xb_ref):
        x = jnp.dot(h_ref[...], win_ref[...], preferred_element_type=jnp.float32)
        x = x + bin_ref[...]
        x_ref[...] = x
        xa_ref[...] = jnp.dot(x, a_ref[...], preferred_element_type=jnp.float32) + be_ref[...]
        xb_ref[...] = jnp.dot(x, b_ref[...], preferred_element_type=jnp.float32)

    grid = (N // BN,)
    blk = pl.BlockSpec((BN, D), lambda i: (i, 0))
    return pl.pallas_call(
        body,
        grid=grid,
        in_specs=[blk, _full_spec((D, D)), _full_spec((1, D)),
                  _full_spec((D, D)), _full_spec((1, D)), _full_spec((D, D))],
        out_specs=[blk, blk, blk],
        out_shape=[jax.ShapeDtypeStruct((N, D), jnp.float32)] * 3,
    )(h, W_in, b_in, A0, be10, B0)


BEC = BE // CHUNK


def _tc_edge(sd, cd, w_r, We2, be2, ne):
    def body(sa_ref, db_ref, cd_ref, wr_ref, w2_ref, b2_ref, m_ref):
        sa = sa_ref[...].reshape(BE, D)
        db = db_ref[...].reshape(BE, D)
        c = cd_ref[...]
        radial = jnp.sum(c * c, axis=1, keepdims=True)
        pre = sa + db + radial * wr_ref[...]
        m1 = _silu(pre)
        m_ref[...] = _silu(
            jnp.dot(m1, w2_ref[...], preferred_element_type=jnp.float32) + b2_ref[...])

    grid = (ne // BE,)
    blk = pl.BlockSpec((BE, D), lambda i: (i, 0))
    return pl.pallas_call(
        body,
        grid=grid,
        in_specs=[pl.BlockSpec((BEC, CHUNK, D), lambda i: (i, 0, 0)),
                  pl.BlockSpec((BEC, CHUNK, D), lambda i: (i, 1, 0)),
                  pl.BlockSpec((BE, 3), lambda i: (i, 0)),
                  _full_spec((1, D)), _full_spec((D, D)), _full_spec((1, D))],
        out_specs=blk,
        out_shape=jax.ShapeDtypeStruct((ne, D), jnp.float32),
    )(sd, sd, cd, w_r, We2, be2)


def _tc_node(x, a0, a1, Wn1x, Wn1a, bn1, Wn2, bn2, An, ben, Bn):
    def body(x_ref, a0_ref, a1_ref, w1x_ref, w1a_ref, b1_ref, w2_ref, b2_ref,
             an_ref, ben_ref, bn_ref, xn_ref, xa_ref, xb_ref):
        x = x_ref[...]
        agg = a0_ref[...] + a1_ref[...]
        hmid = _silu(
            jnp.dot(x, w1x_ref[...], preferred_element_type=jnp.float32)
            + jnp.dot(agg, w1a_ref[...], preferred_element_type=jnp.float32)
            + b1_ref[...])
        xn = jnp.dot(hmid, w2_ref[...], preferred_element_type=jnp.float32) + b2_ref[...]
        xn_ref[...] = xn
        xa_ref[...] = jnp.dot(xn, an_ref[...], preferred_element_type=jnp.float32) + ben_ref[...]
        xb_ref[...] = jnp.dot(xn, bn_ref[...], preferred_element_type=jnp.float32)

    grid = (N // BN,)
    blk = pl.BlockSpec((BN, D), lambda i: (i, 0))
    return pl.pallas_call(
        body,
        grid=grid,
        in_specs=[blk, blk, blk,
                  _full_spec((D, D)), _full_spec((D, D)), _full_spec((1, D)),
                  _full_spec((D, D)), _full_spec((1, D)),
                  _full_spec((D, D)), _full_spec((1, D)), _full_spec((D, D))],
        out_specs=[blk, blk, blk],
        out_shape=[jax.ShapeDtypeStruct((N, D), jnp.float32)] * 3,
    )(x, a0, a1, Wn1x, Wn1a, bn1, Wn2, bn2, An, ben, Bn)


def _tc_node_last(x, a0, a1, Wn1x, Wn1a, bn1, Wn2, bn2, W_out, b_out):
    def body(x_ref, a0_ref, a1_ref, w1x_ref, w1a_ref, b1_ref, w2_ref, b2_ref,
             wo_ref, bo_ref, o_ref):
        x = x_ref[...]
        agg = a0_ref[...] + a1_ref[...]
        hmid = _silu(
            jnp.dot(x, w1x_ref[...], preferred_element_type=jnp.float32)
            + jnp.dot(agg, w1a_ref[...], preferred_element_type=jnp.float32)
            + b1_ref[...])
        xn = jnp.dot(hmid, w2_ref[...], preferred_element_type=jnp.float32) + b2_ref[...]
        o_ref[...] = jnp.dot(xn, wo_ref[...], preferred_element_type=jnp.float32) + bo_ref[...]

    grid = (N // BN,)
    blk = pl.BlockSpec((BN, D), lambda i: (i, 0))
    return pl.pallas_call(
        body,
        grid=grid,
        in_specs=[blk, blk, blk,
                  _full_spec((D, D)), _full_spec((D, D)), _full_spec((1, D)),
                  _full_spec((D, D)), _full_spec((1, D)),
                  _full_spec((D, D)), _full_spec((1, D))],
        out_specs=blk,
        out_shape=jax.ShapeDtypeStruct((N, D), jnp.float32),
    )(x, a0, a1, Wn1x, Wn1a, bn1, Wn2, bn2, W_out, b_out)


def _sc_gather(xa, xb, row2, col2, nreal):
    @functools.partial(
        pl.kernel,
        out_type=jax.ShapeDtypeStruct((nreal, 2 * CHUNK, D), jnp.float32),
        mesh=_sc_mesh(),
        scratch_types=[
            pltpu.VMEM((CPWH, CHUNK), jnp.int32),
            pltpu.VMEM((CPWH, CHUNK), jnp.int32),
            pltpu.VMEM((2, 2 * CHUNK, D), jnp.float32),
            [pltpu.SemaphoreType.DMA] * 2,
            [pltpu.SemaphoreType.DMA] * 2,
            [pltpu.SemaphoreType.DMA] * 2,
        ],
    )
    def k(xa_h, xb_h, row_h, col_h, sd_h, idx_r, idx_c, buf, sgA, sgB, sw):
        w = lax.axis_index("s") * NC + lax.axis_index("c")
        c0 = CPWH * w
        nj = jnp.clip(nreal - c0, 0, CPWH)
        pltpu.sync_copy(row_h.at[pl.ds(c0, CPWH)], idx_r)
        pltpu.sync_copy(col_h.at[pl.ds(c0, CPWH)], idx_c)

        def gath(j, b):
            pltpu.async_copy(xa_h.at[idx_r.at[j]], buf.at[b].at[pl.ds(0, CHUNK)], sgA[b])
            pltpu.async_copy(xb_h.at[idx_c.at[j]], buf.at[b].at[pl.ds(CHUNK, CHUNK)], sgB[b])

        def wait_g(j, b):
            pltpu.make_async_copy(xa_h.at[idx_r.at[j]], buf.at[b].at[pl.ds(0, CHUNK)], sgA[b]).wait()
            pltpu.make_async_copy(xb_h.at[idx_c.at[j]], buf.at[b].at[pl.ds(CHUNK, CHUNK)], sgB[b]).wait()

        def wrb(j, b):
            pltpu.async_copy(buf.at[b], sd_h.at[c0 + j], sw[b])

        def wait_w(j, b):
            pltpu.make_async_copy(buf.at[b], sd_h.at[c0 + j], sw[b]).wait()

        @pl.when(nj > 0)
        def _pipeline():
            gath(0, 0)
            gath(1, 1)
            wait_g(0, 0)
            wrb(0, 0)

            def body(p, carry):
                for b2 in range(2):
                    kk = 2 * p + b2 + 1
                    sb = (b2 + 1) % 2
                    so = 1 - sb
                    wait_g(kk, sb)
                    wrb(kk, sb)
                    wait_w(kk - 1, so)

                    @pl.when(kk + 1 < nj)
                    def _():
                        gath(kk + 1, so)
                return carry

            lax.fori_loop(0, (nj - 1) // 2, body, 0)
            kk = nj - 1
            wait_g(kk, 1)
            wrb(kk, 1)
            wait_w(kk - 1, 0)
            wait_w(kk, 1)

    return k(xa, xb, row2, col2)


HALF0 = 5120
ACC_ROWS = 5136
DUMP = 5128
CPT = 160


def _sc_scatter(m3, row2, nreal):
    @functools.partial(
        pl.kernel,
        out_type=jax.ShapeDtypeStruct((N, D), jnp.float32),
        mesh=_sc_mesh(),
        scratch_types=[
            pltpu.VMEM((CPTH, CHUNK), jnp.int32),
            pltpu.VMEM((CPTH, CHUNK), jnp.int32),
            pltpu.VMEM((2, CHUNK, D), jnp.float32),
            pltpu.VMEM((80, D), jnp.float32),
            pltpu.VMEM_SHARED((ACC_ROWS, D), jnp.float32),
            [pltpu.SemaphoreType.DMA] * 2,
            [pltpu.SemaphoreType.DMA] * 2,
        ],
    )
    def k(m_h, row_h, out_h, idx, idx2, buf, zbuf, acc_sh, sg, sw):
        c = lax.axis_index("c")
        s = lax.axis_index("s")
        base = c * HALF0
        owned = jnp.where(c == 0, HALF0, N - HALF0)
        j0 = CPTH * s
        nj = jnp.clip(nreal - j0, 0, CPTH)
        pltpu.sync_copy(row_h.at[pl.ds(j0, CPTH)], idx)

        zv = jnp.zeros((16,), jnp.float32)

        def zfill(r, carry):
            for g in range(D // 16):
                zbuf[r, g * 16:(g + 1) * 16] = zv
            return carry

        lax.fori_loop(0, 80, zfill, 0)

        def remap(j, carry):
            for g in range(D // 16):
                v = idx[j, g * 16:(g + 1) * 16] - base
                ok = (v >= 0) & (v < owned)
                idx2[j, g * 16:(g + 1) * 16] = jnp.where(ok, v, DUMP)
            return carry

        lax.fori_loop(0, nj, remap, 0)

        tb = pl.multiple_of(s * jnp.where(c == 0, HALF0 // NS, 304), 16)
        n80 = jnp.where(c == 0, 4, 3)

        def zero(t, carry):
            off = pl.multiple_of(tb + 80 * t, 16)
            pltpu.sync_copy(zbuf, acc_sh.at[pl.ds(off, 80)])
            return carry

        lax.fori_loop(0, n80, zero, 0)

        @pl.when(c == 1)
        def _z64():
            off = pl.multiple_of(tb + 240, 8)
            pltpu.sync_copy(zbuf.at[pl.ds(0, 64)], acc_sh.at[pl.ds(off, 64)])

        @pl.when((c == 1) & (s == NS - 1))
        def _ztail():
            pltpu.sync_copy(zbuf.at[pl.ds(0, 16)], acc_sh.at[pl.ds(304 * NS, 16)])

        plsc.subcore_barrier()

        def load(j, b):
            pltpu.async_copy(m_h.at[j0 + j], buf.at[b], sg[b])

        def wait_l(j, b):
            pltpu.make_async_copy(m_h.at[j0 + j], buf.at[b], sg[b]).wait()

        def scat(j, b):
            pltpu.async_copy(buf.at[b], acc_sh.at[idx2.at[j]], sw[b], add=True)

        def wait_s(j, b):
            pltpu.make_async_copy(buf.at[b], acc_sh.at[idx2.at[j]], sw[b]).wait()

        load(0, 0)
        load(1, 1)
        wait_l(0, 0)
        scat(0, 0)

        def body(p, carry):
            for b2 in range(2):
                kk = 2 * p + b2 + 1
                sb = (b2 + 1) % 2
                so = 1 - sb
                wait_l(kk, sb)
                scat(kk, sb)
                wait_s(kk - 1, so)

                @pl.when(kk + 1 < nj)
                def _():
                    load(kk + 1, so)
            return carry

        lax.fori_loop(0, (nj - 1) // 2, body, 0)
        kk = nj - 1
        wait_l(kk, 1)
        scat(kk, 1)
        wait_s(kk - 1, 0)
        wait_s(kk, 1)
        plsc.subcore_barrier()

        @pl.when(c == 0)
        def _wb0():
            off = pl.multiple_of((HALF0 // NS) * s, 8)
            pltpu.sync_copy(acc_sh.at[pl.ds(off, HALF0 // NS)],
                            out_h.at[pl.ds(off, HALF0 // NS)])

        @pl.when(c == 1)
        def _wb1():
            off = pl.multiple_of(304 * s, 8)
            pltpu.sync_copy(acc_sh.at[pl.ds(off, 304)],
                            out_h.at[pl.ds(HALF0 + off, 304)])

        @pl.when((c == 1) & (s == NS - 1))
        def _wb1tail():
            pltpu.sync_copy(acc_sh.at[pl.ds(304 * NS, 16)],
                            out_h.at[pl.ds(HALF0 + 304 * NS, 16)])

    return k(m3, row2)


def kernel(h, edge_index, coord_diff, W_in, b_in, W_out, b_out,
           We1, be1, We2, be2, Wn1, bn1, Wn2, bn2):
    row = edge_index[0].astype(jnp.int32)
    col = edge_index[1].astype(jnp.int32)
    pad = ((0, NCHUNK_PAD - NCHUNK), (0, 0))
    spread = (jnp.arange((NCHUNK_PAD - NCHUNK) * CHUNK, dtype=jnp.int32)
              .reshape(NCHUNK_PAD - NCHUNK, CHUNK) * 79) % N
    row2g = jnp.concatenate([row.reshape(NCHUNK, CHUNK), spread])
    col2g = jnp.concatenate([col.reshape(NCHUNK, CHUNK), spread])
    row2s = jnp.pad(row.reshape(NCHUNK, CHUNK), pad, constant_values=N)

    rowH = (row2g[:HCH], row2g[HCH:])
    colH = (col2g[:HCH], col2g[HCH:])
    rowHs = (row2s[:HCH], row2s[HCH:])
    cdH = (coord_diff[:EH], coord_diff[EH:])
    b_in_r = b_in.reshape(1, D)
    b_out_r = b_out.reshape(1, D)

    x, xa, xb = _tc_prologue(
        h, W_in, b_in_r, We1[0, :D, :], be1[0].reshape(1, D), We1[0, D:2 * D, :])

    out = None
    for l in range(L):
        w_r = We1[l, 2 * D, :].reshape(1, D)
        be2_r = be2[l].reshape(1, D)
        sd0 = _sc_gather(xa, xb, rowH[0], colH[0], HCH)
        m0 = _tc_edge(sd0, cdH[0], w_r, We2[l], be2_r, EH)
        sd1 = _sc_gather(xa, xb, rowH[1], colH[1], NCH1)
        a0 = _sc_scatter(m0.reshape(HCH, CHUNK, D), rowHs[0], HCH)
        m1 = _tc_edge(sd1, cdH[1], w_r, We2[l], be2_r, NCH1 * CHUNK)
        a1 = _sc_scatter(m1.reshape(NCH1, CHUNK, D), rowHs[1], NCH1)
        if l < L - 1:
            x, xa, xb = _tc_node(
                x, a0, a1, Wn1[l, :D, :], Wn1[l, D:, :], bn1[l].reshape(1, D),
                Wn2[l], bn2[l].reshape(1, D),
                We1[l + 1, :D, :], be1[l + 1].reshape(1, D), We1[l + 1, D:2 * D, :])
        else:
            out = _tc_node_last(
                x, a0, a1, Wn1[l, :D, :], Wn1[l, D:, :], bn1[l].reshape(1, D),
                Wn2[l], bn2[l].reshape(1, D), W_out, b_out_r)
    return out
